# baseline (speedup 1.0000x reference)
import functools
import math

import jax
import jax.numpy as jnp
from jax import lax
from jax.experimental import pallas as pl
from jax.experimental.pallas import tpu as pltpu
from jax.experimental.pallas import tpu_sc as plsc

NC = 2
NS = 16
L = 16
NW = NC * NS

R = 4096
S = 200
D = 64
RPW = R // NW
HALVES = ((0, 104), (104, 96))
NG = 4
NGROUP = RPW * 2 // NG
SCALE = math.sqrt(D)

_mesh = plsc.VectorSubcoreMesh(core_axis_name="c", subcore_axis_name="s")


@functools.partial(
    pl.kernel,
    out_type=jax.ShapeDtypeStruct((R, S, D), jnp.float32),
    mesh=_mesh,
    scratch_types=[
        pltpu.VMEM((RPW, S), jnp.int32),
        pltpu.VMEM((NG, 104, D), jnp.float32),
        pltpu.VMEM((NG, 104, D), jnp.float32),
    ]
    + [pltpu.SemaphoreType.DMA] * (2 * NG),
    compiler_params=pltpu.CompilerParams(use_tc_tiling_on_sc=False),
)
def _embed(x_hbm, table_hbm, out_hbm, idx_v, g_v, o_v, *sems):
    gsem, osem = sems[:NG], sems[NG:]
    wid = lax.axis_index("s") * NC + lax.axis_index("c")
    row0 = wid * RPW
    pltpu.sync_copy(x_hbm.at[pl.ds(row0, RPW)], idx_v)

    for b in range(NG):
        off, ln = HALVES[b % 2]
        pltpu.async_copy(
            table_hbm.at[idx_v.at[b // 2, pl.ds(off, ln)]],
            g_v.at[b, pl.ds(0, ln)],
            gsem[b],
        )

    def group(g, carry):
        r2 = 2 * g
        for b in range(NG):
            off, ln = HALVES[b % 2]
            r_loc = r2 + b // 2
            pltpu.make_async_copy(
                table_hbm.at[idx_v.at[r_loc, pl.ds(off, ln)]],
                g_v.at[b, pl.ds(0, ln)],
                gsem[b],
            ).wait()

            @pl.when(g > 0)
            def _():
                pltpu.make_async_copy(
                    o_v.at[b, pl.ds(0, ln)],
                    out_hbm.at[0, pl.ds(off, ln)],
                    osem[b],
                ).wait()

            def srow(k, c2, b=b):
                r0 = k * 8
                for dr in range(8):
                    for c in range(D // L):
                        sl = pl.ds(c * L, L)
                        o_v[b, r0 + dr, sl] = g_v[b, r0 + dr, sl] * SCALE
                return c2

            lax.fori_loop(0, ln // 8, srow, 0)

            pltpu.async_copy(
                o_v.at[b, pl.ds(0, ln)],
                out_hbm.at[row0 + r_loc, pl.ds(off, ln)],
                osem[b],
            )

            @pl.when(g < NGROUP - 1)
            def _():
                pltpu.async_copy(
                    table_hbm.at[idx_v.at[r_loc + 2, pl.ds(off, ln)]],
                    g_v.at[b, pl.ds(0, ln)],
                    gsem[b],
                )
        return carry

    lax.fori_loop(0, NGROUP, group, 0)

    for b in range(NG):
        off, ln = HALVES[b % 2]
        pltpu.make_async_copy(
            o_v.at[b, pl.ds(0, ln)],
            out_hbm.at[0, pl.ds(off, ln)],
            osem[b],
        ).wait()


def kernel(x, table):
    return _embed(x, table)
